# baseline (speedup 1.0000x reference)
import functools

import jax
import jax.numpy as jnp
import numpy as np
from jax import lax
from jax.experimental import pallas as pl
from jax.experimental.pallas import tpu as pltpu
from jax.experimental.pallas import tpu_sc as plsc

NC = 2
NS = 16
NW = NC * NS
L = 16
CHUNK = 80

N_PAD = 10240
ROWS_PER_TILE = N_PAD // NS


def _sc_mesh():
    return plsc.VectorSubcoreMesh(
        core_axis_name="c", subcore_axis_name="s", num_cores=NC,
        num_subcores=NS)


def _make_deg_kernel(cpw):
    @functools.partial(
        pl.kernel,
        out_type=jax.ShapeDtypeStruct((NC, N_PAD), jnp.float32),
        mesh=_sc_mesh(),
        scratch_types=[
            pltpu.VMEM((cpw, CHUNK), jnp.int32),
            pltpu.VMEM((cpw, CHUNK), jnp.float32),
            pltpu.VMEM((ROWS_PER_TILE,), jnp.float32),
            pltpu.VMEM_SHARED((N_PAD,), jnp.float32),
            pltpu.SemaphoreType.DMA,
        ],
    )
    def deg_kernel(ei_hbm, ew_hbm, out_hbm, idx_v, ew_v, z_v, acc_sh, dsem):
        c = lax.axis_index("c")
        s = lax.axis_index("s")
        w = c * NS + s

        def zfill(i, _):
            z_v[pl.ds(i * L, L)] = jnp.zeros((L,), jnp.float32)
            return 0
        lax.fori_loop(0, ROWS_PER_TILE // L, zfill, 0)
        pltpu.sync_copy(z_v, acc_sh.at[pl.ds(s * ROWS_PER_TILE,
                                             ROWS_PER_TILE)])
        plsc.subcore_barrier()

        pltpu.sync_copy(ei_hbm.at[1, w], idx_v)
        pltpu.sync_copy(ew_hbm.at[w], ew_v)

        def body(k, _):
            pltpu.make_async_copy(
                ew_v.at[k], acc_sh.at[idx_v.at[k]], dsem).start(add=True)
            return 0
        lax.fori_loop(0, cpw, body, 0)

        def drain(k, _):
            pltpu.make_async_copy(
                ew_v.at[k], acc_sh.at[idx_v.at[k]], dsem).wait()
            return 0
        lax.fori_loop(0, cpw, drain, 0)

        plsc.subcore_barrier()
        pltpu.sync_copy(acc_sh.at[pl.ds(s * ROWS_PER_TILE, ROWS_PER_TILE)],
                        out_hbm.at[c, pl.ds(s * ROWS_PER_TILE,
                                            ROWS_PER_TILE)])

    return deg_kernel


NBUF = 5


def _make_agg_kernel(cpw, width, n_rows, stage_table, out_w=None):
    out_w = width if out_w is None else out_w
    assert cpw % NBUF == 0

    scratch = [
        pltpu.VMEM((cpw, CHUNK), jnp.int32),
        pltpu.VMEM((cpw, CHUNK), jnp.int32),
        pltpu.VMEM((cpw, CHUNK), jnp.float32),
        [pltpu.VMEM((CHUNK, width), jnp.float32) for _ in range(NBUF)],
        [pltpu.VMEM((CHUNK, width), jnp.float32) for _ in range(NBUF)],
        pltpu.VMEM_SHARED((N_PAD, width), jnp.float32),
    ]
    if stage_table:
        scratch.append(pltpu.VMEM_SHARED((n_rows, width), jnp.float32))
    scratch += [
        [pltpu.SemaphoreType.DMA for _ in range(NBUF)],
        [pltpu.SemaphoreType.DMA for _ in range(NBUF)],
    ]

    @functools.partial(
        pl.kernel,
        out_type=jax.ShapeDtypeStruct((NC, N_PAD, out_w), jnp.float32),
        mesh=_sc_mesh(),
        scratch_types=scratch,
        compiler_params=pltpu.CompilerParams(use_tc_tiling_on_sc=False),
    )
    def agg_kernel(table_hbm, ei_hbm, ew_hbm, out_hbm,
                   src_v, dst_v, ew_v, gbufs, sbufs, acc_sh, *rest):
        if stage_table:
            tab_sh, gsems, ssems = rest
            gsrc = tab_sh
        else:
            gsems, ssems = rest
            gsrc = table_hbm
        c = lax.axis_index("c")
        s = lax.axis_index("s")
        w = c * NS + s
        rounds = cpw // NBUF

        for g in range(CHUNK):
            for j in range(width // L):
                gbufs[0][g, pl.ds(j * L, L)] = jnp.zeros((L,), jnp.float32)
        for i in range(ROWS_PER_TILE // CHUNK):
            pltpu.sync_copy(
                gbufs[0],
                acc_sh.at[pl.ds(s * ROWS_PER_TILE + i * CHUNK, CHUNK)])
        if stage_table:
            tab_rows = n_rows // NS
            pltpu.sync_copy(
                table_hbm.at[pl.ds(s * tab_rows, tab_rows)],
                tab_sh.at[pl.ds(s * tab_rows, tab_rows)])
        plsc.subcore_barrier()

        pltpu.sync_copy(ei_hbm.at[0, w], src_v)
        pltpu.sync_copy(ei_hbm.at[1, w], dst_v)
        pltpu.sync_copy(ew_hbm.at[w], ew_v)

        def gather(k, j):
            pltpu.make_async_copy(
                gsrc.at[src_v.at[k]], gbufs[j], gsems[j]).start()

        def scale(k, j):
            for g in range(CHUNK // L):
                ewv = ew_v[k, pl.ds(g * L, L)]
                bcs = [
                    jnp.take_along_axis(
                        ewv, jnp.full((L,), i, jnp.int32), axis=0)
                    for i in range(L)
                ]
                for i in range(L):
                    row = g * L + i
                    for jj in range(width // L):
                        sl = gbufs[j][row, pl.ds(jj * L, L)]
                        sbufs[j][row, pl.ds(jj * L, L)] = sl * bcs[i]

        for j in range(NBUF):
            gather(j, j)

        def body(i, _):
            for j in range(NBUF):
                k = i * NBUF + j
                pltpu.make_async_copy(
                    gsrc.at[src_v.at[k]], gbufs[j], gsems[j]).wait()

                @pl.when(i > 0)
                def _():
                    pltpu.make_async_copy(
                        sbufs[j], acc_sh.at[dst_v.at[k - NBUF]],
                        ssems[j]).wait()

                scale(k, j)

                @pl.when(i + 1 < rounds)
                def _():
                    gather(k + NBUF, j)

                pltpu.make_async_copy(
                    sbufs[j], acc_sh.at[dst_v.at[k]], ssems[j]
                ).start(add=True)
            return 0
        lax.fori_loop(0, rounds, body, 0)

        for j in range(NBUF):
            k = cpw - NBUF + j
            pltpu.make_async_copy(
                sbufs[j], acc_sh.at[dst_v.at[k]], ssems[j]).wait()

        plsc.subcore_barrier()
        if out_w == width:
            src = acc_sh.at[pl.ds(s * ROWS_PER_TILE, ROWS_PER_TILE)]
        else:
            src = acc_sh.at[pl.ds(s * ROWS_PER_TILE, ROWS_PER_TILE),
                            pl.ds(0, out_w)]
        pltpu.sync_copy(
            src, out_hbm.at[c, pl.ds(s * ROWS_PER_TILE, ROWS_PER_TILE)])

    return agg_kernel


def _tc_prep(deg_p, x, W1, n):
    h = W1.shape[1]

    def body(degp_ref, x_ref, w1_ref, xws_ref, dis_ref):
        dp = degp_ref[...]
        deg = dp[0] + dp[1] + 1.0
        dis = lax.rsqrt(deg)[:n][:, None]
        xw = jnp.dot(x_ref[...], w1_ref[...],
                     preferred_element_type=jnp.float32)
        xws_ref[...] = xw * dis
        dis_ref[...] = dis

    return pl.pallas_call(
        body,
        out_shape=(jax.ShapeDtypeStruct((n, h), jnp.float32),
                   jax.ShapeDtypeStruct((n, 1), jnp.float32)),
    )(deg_p, x, W1)


def _tc_mid(s1_p, xws, dis, b1, W2p, n):
    wpad = W2p.shape[1]

    def body(s1p_ref, xws_ref, dis_ref, b1_ref, w2_ref, hws_ref):
        dis_c = dis_ref[...]
        s1 = s1p_ref[0, :n] + s1p_ref[1, :n]
        out1 = (s1 + xws_ref[...]) * dis_c + b1_ref[...]
        h = jnp.maximum(out1, 0.0)
        hw = jnp.dot(h, w2_ref[...], preferred_element_type=jnp.float32)
        hws_ref[...] = hw * dis_c

    return pl.pallas_call(
        body,
        out_shape=jax.ShapeDtypeStruct((n, wpad), jnp.float32),
    )(s1_p, xws, dis, b1, W2p)


def _tc_final(s2_p, hws, dis, b2, n, out_dim):
    def body(s2p_ref, hws_ref, dis_ref, b2_ref, out_ref):
        s2 = s2p_ref[0, :n, :out_dim] + s2p_ref[1, :n, :out_dim]
        out_ref[...] = ((s2 + hws_ref[:, :out_dim]) * dis_ref[...]
                        + b2_ref[...])

    return pl.pallas_call(
        body,
        out_shape=jax.ShapeDtypeStruct((n, out_dim), jnp.float32),
    )(s2_p, hws, dis, b2)


def kernel(x, edge_index, edge_weight, W1, b1, W2, b2):
    n, d = x.shape
    e = edge_index.shape[1]
    h = W1.shape[1]
    out_dim = W2.shape[1]
    w2pad = 16

    assert e % (NW * CHUNK) == 0
    cpw = e // (NW * CHUNK)
    assert cpw % NBUF == 0
    ei = edge_index.astype(jnp.int32).reshape(2, NW, cpw, CHUNK)
    ew_p = edge_weight.astype(jnp.float32).reshape(NW, cpw, CHUNK)

    deg_p = _make_deg_kernel(cpw)(ei, ew_p)
    xws, dis = _tc_prep(deg_p, x, W1, n)
    s1_p = _make_agg_kernel(cpw, h, n, False)(xws, ei, ew_p)
    W2p = jnp.concatenate(
        [W2, jnp.zeros((h, w2pad - out_dim), jnp.float32)], axis=1)
    hws = _tc_mid(s1_p, xws, dis, b1.reshape(1, h), W2p, n)
    s2_p = _make_agg_kernel(cpw, w2pad, n, True, 8)(hws, ei, ew_p)
    return _tc_final(s2_p, hws, dis, b2.reshape(1, out_dim), n, out_dim)


# speedup vs baseline: 1.0375x; 1.0375x over previous
import functools

import jax
import jax.numpy as jnp
import numpy as np
from jax import lax
from jax.experimental import pallas as pl
from jax.experimental.pallas import tpu as pltpu
from jax.experimental.pallas import tpu_sc as plsc

NC = 2
NS = 16
NW = NC * NS
L = 16
CHUNK = 80

N_PAD = 10240
ROWS_PER_TILE = N_PAD // NS


def _sc_mesh():
    return plsc.VectorSubcoreMesh(
        core_axis_name="c", subcore_axis_name="s", num_cores=NC,
        num_subcores=NS)


def _make_deg_kernel(cpw):
    @functools.partial(
        pl.kernel,
        out_type=jax.ShapeDtypeStruct((NC, N_PAD), jnp.float32),
        mesh=_sc_mesh(),
        scratch_types=[
            pltpu.VMEM((cpw, CHUNK), jnp.int32),
            pltpu.VMEM((cpw, CHUNK), jnp.float32),
            pltpu.VMEM((ROWS_PER_TILE,), jnp.float32),
            pltpu.VMEM_SHARED((N_PAD,), jnp.float32),
            pltpu.SemaphoreType.DMA,
        ],
    )
    def deg_kernel(ei_hbm, ew_hbm, out_hbm, idx_v, ew_v, z_v, acc_sh, dsem):
        c = lax.axis_index("c")
        s = lax.axis_index("s")
        w = c * NS + s

        def zfill(i, _):
            z_v[pl.ds(i * L, L)] = jnp.zeros((L,), jnp.float32)
            return 0
        lax.fori_loop(0, ROWS_PER_TILE // L, zfill, 0)
        pltpu.sync_copy(z_v, acc_sh.at[pl.ds(s * ROWS_PER_TILE,
                                             ROWS_PER_TILE)])
        plsc.subcore_barrier()

        pltpu.sync_copy(ei_hbm.at[1, w], idx_v)
        pltpu.sync_copy(ew_hbm.at[w], ew_v)

        def body(k, _):
            pltpu.make_async_copy(
                ew_v.at[k], acc_sh.at[idx_v.at[k]], dsem).start(add=True)
            return 0
        lax.fori_loop(0, cpw, body, 0)

        def drain(k, _):
            pltpu.make_async_copy(
                ew_v.at[k], acc_sh.at[idx_v.at[k]], dsem).wait()
            return 0
        lax.fori_loop(0, cpw, drain, 0)

        plsc.subcore_barrier()
        pltpu.sync_copy(acc_sh.at[pl.ds(s * ROWS_PER_TILE, ROWS_PER_TILE)],
                        out_hbm.at[c, pl.ds(s * ROWS_PER_TILE,
                                            ROWS_PER_TILE)])

    return deg_kernel


NBUF = 5


def _make_agg_kernel(cpw, width, n_rows, stage_table, out_w=None):
    out_w = width if out_w is None else out_w
    assert cpw % NBUF == 0

    scratch = [
        pltpu.VMEM((cpw, CHUNK), jnp.int32),
        pltpu.VMEM((cpw, CHUNK), jnp.int32),
        pltpu.VMEM((cpw, CHUNK), jnp.float32),
        [pltpu.VMEM((CHUNK, width), jnp.float32) for _ in range(NBUF)],
        [pltpu.VMEM((CHUNK, width), jnp.float32) for _ in range(NBUF)],
        pltpu.VMEM_SHARED((N_PAD, width), jnp.float32),
    ]
    if stage_table:
        scratch.append(pltpu.VMEM_SHARED((n_rows, width), jnp.float32))
    scratch += [
        [pltpu.SemaphoreType.DMA for _ in range(NBUF)],
        [pltpu.SemaphoreType.DMA for _ in range(NBUF)],
    ]

    @functools.partial(
        pl.kernel,
        out_type=jax.ShapeDtypeStruct((NC, N_PAD, out_w), jnp.float32),
        mesh=_sc_mesh(),
        scratch_types=scratch,
        compiler_params=pltpu.CompilerParams(use_tc_tiling_on_sc=False),
    )
    def agg_kernel(table_hbm, ei_hbm, ew_hbm, out_hbm,
                   src_v, dst_v, ew_v, gbufs, sbufs, acc_sh, *rest):
        if stage_table:
            tab_sh, gsems, ssems = rest
            gsrc = tab_sh
        else:
            gsems, ssems = rest
            gsrc = table_hbm
        c = lax.axis_index("c")
        s = lax.axis_index("s")
        w = c * NS + s
        rounds = cpw // NBUF

        for g in range(CHUNK):
            for j in range(width // L):
                gbufs[0][g, pl.ds(j * L, L)] = jnp.zeros((L,), jnp.float32)
        for i in range(ROWS_PER_TILE // CHUNK):
            pltpu.sync_copy(
                gbufs[0],
                acc_sh.at[pl.ds(s * ROWS_PER_TILE + i * CHUNK, CHUNK)])
        if stage_table:
            tab_rows = n_rows // NS
            pltpu.sync_copy(
                table_hbm.at[pl.ds(s * tab_rows, tab_rows)],
                tab_sh.at[pl.ds(s * tab_rows, tab_rows)])
        plsc.subcore_barrier()

        pltpu.sync_copy(ei_hbm.at[0, w], src_v)
        pltpu.sync_copy(ei_hbm.at[1, w], dst_v)
        pltpu.sync_copy(ew_hbm.at[w], ew_v)

        def gather(k, j):
            pltpu.make_async_copy(
                gsrc.at[src_v.at[k]], gbufs[j], gsems[j]).start()

        def scale(k, j):
            for g in range(CHUNK // L):
                ewv = ew_v[k, pl.ds(g * L, L)]
                bcs = [
                    jnp.take_along_axis(
                        ewv, jnp.full((L,), i, jnp.int32), axis=0)
                    for i in range(L)
                ]
                for i in range(L):
                    row = g * L + i
                    for jj in range(width // L):
                        sl = gbufs[j][row, pl.ds(jj * L, L)]
                        sbufs[j][row, pl.ds(jj * L, L)] = sl * bcs[i]

        for j in range(NBUF):
            gather(j, j)

        def body(i, _):
            for j in range(NBUF):
                k = i * NBUF + j
                pltpu.make_async_copy(
                    gsrc.at[src_v.at[k]], gbufs[j], gsems[j]).wait()

                @pl.when(i > 0)
                def _():
                    pltpu.make_async_copy(
                        sbufs[j], acc_sh.at[dst_v.at[k - NBUF]],
                        ssems[j]).wait()

                scale(k, j)

                @pl.when(i + 1 < rounds)
                def _():
                    gather(k + NBUF, j)

                pltpu.make_async_copy(
                    sbufs[j], acc_sh.at[dst_v.at[k]], ssems[j]
                ).start(add=True)
            return 0
        lax.fori_loop(0, rounds, body, 0)

        for j in range(NBUF):
            k = cpw - NBUF + j
            pltpu.make_async_copy(
                sbufs[j], acc_sh.at[dst_v.at[k]], ssems[j]).wait()

        plsc.subcore_barrier()
        if out_w == width:
            src = acc_sh.at[pl.ds(s * ROWS_PER_TILE, ROWS_PER_TILE)]
        else:
            src = acc_sh.at[pl.ds(s * ROWS_PER_TILE, ROWS_PER_TILE),
                            pl.ds(0, out_w)]
        pltpu.sync_copy(
            src, out_hbm.at[c, pl.ds(s * ROWS_PER_TILE, ROWS_PER_TILE)])

    return agg_kernel


def _tc_prep(deg_p, x, W1, n):
    h = W1.shape[1]

    def body(degp_ref, x_ref, w1_ref, xws_ref, dis_ref):
        dp = degp_ref[...]
        deg = dp[0] + dp[1] + 1.0
        dis = lax.rsqrt(deg)[:n][:, None]
        xw = jnp.dot(x_ref[...], w1_ref[...],
                     preferred_element_type=jnp.float32)
        xws_ref[...] = xw * dis
        dis_ref[...] = dis

    return pl.pallas_call(
        body,
        out_shape=(jax.ShapeDtypeStruct((n, h), jnp.float32),
                   jax.ShapeDtypeStruct((n, 1), jnp.float32)),
    )(deg_p, x, W1)


def _tc_mid(s1_p, xws, dis, b1, W2p, n):
    wpad = W2p.shape[1]

    def body(s1p_ref, xws_ref, dis_ref, b1_ref, w2_ref, hws_ref):
        dis_c = dis_ref[...]
        s1 = s1p_ref[0, :n] + s1p_ref[1, :n]
        out1 = (s1 + xws_ref[...]) * dis_c + b1_ref[...]
        h = jnp.maximum(out1, 0.0)
        hw = jnp.dot(h, w2_ref[...], preferred_element_type=jnp.float32)
        hws_ref[...] = hw * dis_c

    return pl.pallas_call(
        body,
        out_shape=jax.ShapeDtypeStruct((n, wpad), jnp.float32),
    )(s1_p, xws, dis, b1, W2p)


def _tc_final(s2_p, hws, dis, b2, n, out_dim):
    def body(s2p_ref, hws_ref, dis_ref, b2_ref, out_ref):
        s2 = s2p_ref[0, :n, :out_dim] + s2p_ref[1, :n, :out_dim]
        out_ref[...] = ((s2 + hws_ref[:, :out_dim]) * dis_ref[...]
                        + b2_ref[...])

    return pl.pallas_call(
        body,
        out_shape=jax.ShapeDtypeStruct((n, out_dim), jnp.float32),
    )(s2_p, hws, dis, b2)


def kernel(x, edge_index, edge_weight, W1, b1, W2, b2):
    n, d = x.shape
    e = edge_index.shape[1]
    h = W1.shape[1]
    out_dim = W2.shape[1]
    w2pad = 16

    assert e % (NW * CHUNK) == 0
    cpw = e // (NW * CHUNK)
    assert cpw % NBUF == 0
    ei = edge_index.astype(jnp.int32).reshape(2, NW, cpw, CHUNK)
    ew_p = edge_weight.astype(jnp.float32).reshape(NW, cpw, CHUNK)

    deg_p = _make_deg_kernel(cpw)(ei, ew_p)
    xws, dis = _tc_prep(deg_p, x, W1, n)
    s1_p = _make_agg_kernel(cpw, h, n, False)(xws, ei, ew_p)
    W2p = jnp.concatenate(
        [W2, jnp.zeros((h, w2pad - out_dim), jnp.float32)], axis=1)
    hws = _tc_mid(s1_p, xws, dis, b1.reshape(1, h), W2p, n)
    s2_p = _make_agg_kernel(cpw, w2pad, n, True)(hws, ei, ew_p)
    return _tc_final(s2_p, hws, dis, b2.reshape(1, out_dim), n, out_dim)


# speedup vs baseline: 1.0395x; 1.0019x over previous
import functools

import jax
import jax.numpy as jnp
from jax import lax
from jax.experimental import pallas as pl
from jax.experimental.pallas import tpu as pltpu
from jax.experimental.pallas import tpu_sc as plsc

NC = 2
NS = 16
NW = NC * NS
L = 16
CHUNK = 80

N_PAD = 10240
ROWS_PER_TILE = N_PAD // NS


def _sc_mesh():
    return plsc.VectorSubcoreMesh(
        core_axis_name="c", subcore_axis_name="s", num_cores=NC,
        num_subcores=NS)


def _make_deg_kernel(cpw):
    @functools.partial(
        pl.kernel,
        out_type=jax.ShapeDtypeStruct((NC, N_PAD), jnp.float32),
        mesh=_sc_mesh(),
        scratch_types=[
            pltpu.VMEM((cpw, CHUNK), jnp.int32),
            pltpu.VMEM((cpw, CHUNK), jnp.float32),
            pltpu.VMEM((ROWS_PER_TILE,), jnp.float32),
            pltpu.VMEM_SHARED((N_PAD,), jnp.float32),
            pltpu.SemaphoreType.DMA,
        ],
    )
    def deg_kernel(ei_hbm, ew_hbm, out_hbm, idx_v, ew_v, z_v, acc_sh, dsem):
        c = lax.axis_index("c")
        s = lax.axis_index("s")
        w = c * NS + s

        def zfill(i, _):
            z_v[pl.ds(i * L, L)] = jnp.zeros((L,), jnp.float32)
            return 0
        lax.fori_loop(0, ROWS_PER_TILE // L, zfill, 0)
        pltpu.sync_copy(z_v, acc_sh.at[pl.ds(s * ROWS_PER_TILE,
                                             ROWS_PER_TILE)])
        plsc.subcore_barrier()

        pltpu.sync_copy(ei_hbm.at[1, w], idx_v)
        pltpu.sync_copy(ew_hbm.at[w], ew_v)

        def body(k, _):
            pltpu.make_async_copy(
                ew_v.at[k], acc_sh.at[idx_v.at[k]], dsem).start(add=True)
            return 0
        lax.fori_loop(0, cpw, body, 0)

        def drain(k, _):
            pltpu.make_async_copy(
                ew_v.at[k], acc_sh.at[idx_v.at[k]], dsem).wait()
            return 0
        lax.fori_loop(0, cpw, drain, 0)

        plsc.subcore_barrier()
        pltpu.sync_copy(acc_sh.at[pl.ds(s * ROWS_PER_TILE, ROWS_PER_TILE)],
                        out_hbm.at[c, pl.ds(s * ROWS_PER_TILE,
                                            ROWS_PER_TILE)])

    return deg_kernel


NBUF = 5


def _make_agg_kernel(cpw, width, n_rows, stage_table, out_w=None):
    out_w = width if out_w is None else out_w
    assert cpw % NBUF == 0

    scratch = [
        pltpu.VMEM((cpw, CHUNK), jnp.int32),
        pltpu.VMEM((cpw, CHUNK), jnp.int32),
        pltpu.VMEM((cpw, CHUNK), jnp.float32),
        [pltpu.VMEM((CHUNK, width), jnp.float32) for _ in range(NBUF)],
        [pltpu.VMEM((CHUNK, width), jnp.float32) for _ in range(NBUF)],
        pltpu.VMEM_SHARED((N_PAD, width), jnp.float32),
    ]
    if stage_table:
        scratch.append(pltpu.VMEM_SHARED((n_rows, width), jnp.float32))
    scratch += [
        [pltpu.SemaphoreType.DMA for _ in range(NBUF)],
        [pltpu.SemaphoreType.DMA for _ in range(NBUF)],
    ]

    @functools.partial(
        pl.kernel,
        out_type=jax.ShapeDtypeStruct((NC, N_PAD, out_w), jnp.float32),
        mesh=_sc_mesh(),
        scratch_types=scratch,
        compiler_params=pltpu.CompilerParams(use_tc_tiling_on_sc=False),
    )
    def agg_kernel(table_hbm, ei_hbm, ew_hbm, out_hbm,
                   src_v, dst_v, ew_v, gbufs, sbufs, acc_sh, *rest):
        if stage_table:
            tab_sh, gsems, ssems = rest
            gsrc = tab_sh
        else:
            gsems, ssems = rest
            gsrc = table_hbm
        c = lax.axis_index("c")
        s = lax.axis_index("s")
        w = c * NS + s
        rounds = cpw // NBUF

        for g in range(CHUNK):
            for j in range(width // L):
                gbufs[0][g, pl.ds(j * L, L)] = jnp.zeros((L,), jnp.float32)
        for i in range(ROWS_PER_TILE // CHUNK):
            pltpu.sync_copy(
                gbufs[0],
                acc_sh.at[pl.ds(s * ROWS_PER_TILE + i * CHUNK, CHUNK)])
        if stage_table:
            tab_rows = n_rows // NS
            pltpu.sync_copy(
                table_hbm.at[pl.ds(s * tab_rows, tab_rows)],
                tab_sh.at[pl.ds(s * tab_rows, tab_rows)])
        plsc.subcore_barrier()

        pltpu.sync_copy(ei_hbm.at[0, w], src_v)
        pltpu.sync_copy(ei_hbm.at[1, w], dst_v)
        pltpu.sync_copy(ew_hbm.at[w], ew_v)

        def gather(k, j):
            pltpu.make_async_copy(
                gsrc.at[src_v.at[k]], gbufs[j], gsems[j]).start()

        def scale(k, j):
            for g in range(CHUNK // L):
                ewv = ew_v[k, pl.ds(g * L, L)]
                bcs = [
                    jnp.take_along_axis(
                        ewv, jnp.full((L,), i, jnp.int32), axis=0)
                    for i in range(L)
                ]
                for i in range(L):
                    row = g * L + i
                    for jj in range(width // L):
                        sl = gbufs[j][row, pl.ds(jj * L, L)]
                        sbufs[j][row, pl.ds(jj * L, L)] = sl * bcs[i]

        for j in range(NBUF):
            gather(j, j)

        def body(i, _):
            for j in range(NBUF):
                k = i * NBUF + j
                pltpu.make_async_copy(
                    gsrc.at[src_v.at[k]], gbufs[j], gsems[j]).wait()

                @pl.when(i > 0)
                def _():
                    pltpu.make_async_copy(
                        sbufs[j], acc_sh.at[dst_v.at[k - NBUF]],
                        ssems[j]).wait()

                scale(k, j)

                @pl.when(i + 1 < rounds)
                def _():
                    gather(k + NBUF, j)

                pltpu.make_async_copy(
                    sbufs[j], acc_sh.at[dst_v.at[k]], ssems[j]
                ).start(add=True)
            return 0
        lax.fori_loop(0, rounds, body, 0)

        for j in range(NBUF):
            k = cpw - NBUF + j
            pltpu.make_async_copy(
                sbufs[j], acc_sh.at[dst_v.at[k]], ssems[j]).wait()

        plsc.subcore_barrier()
        if out_w == width:
            src = acc_sh.at[pl.ds(s * ROWS_PER_TILE, ROWS_PER_TILE)]
        else:
            src = acc_sh.at[pl.ds(s * ROWS_PER_TILE, ROWS_PER_TILE),
                            pl.ds(0, out_w)]
        pltpu.sync_copy(
            src, out_hbm.at[c, pl.ds(s * ROWS_PER_TILE, ROWS_PER_TILE)])

    return agg_kernel


def _tc_prep(deg_p, x, W1, n):
    h = W1.shape[1]

    def body(degp_ref, x_ref, w1_ref, xws_ref, dis_ref):
        dp = degp_ref[...]
        deg = dp[0] + dp[1] + 1.0
        dis = lax.rsqrt(deg)[:n][:, None]
        xw = jnp.dot(x_ref[...], w1_ref[...],
                     preferred_element_type=jnp.float32)
        xws_ref[...] = xw * dis
        dis_ref[...] = dis

    return pl.pallas_call(
        body,
        out_shape=(jax.ShapeDtypeStruct((n, h), jnp.float32),
                   jax.ShapeDtypeStruct((n, 1), jnp.float32)),
    )(deg_p, x, W1)


def _tc_mid(s1_p, xws, dis, b1, W2p, n):
    wpad = W2p.shape[1]

    def body(s1p_ref, xws_ref, dis_ref, b1_ref, w2_ref, hws_ref):
        dis_c = dis_ref[...]
        s1 = s1p_ref[0, :n] + s1p_ref[1, :n]
        out1 = (s1 + xws_ref[...]) * dis_c + b1_ref[...]
        h = jnp.maximum(out1, 0.0)
        hw = jnp.dot(h, w2_ref[...], preferred_element_type=jnp.float32)
        hws_ref[...] = hw * dis_c

    return pl.pallas_call(
        body,
        out_shape=jax.ShapeDtypeStruct((n, wpad), jnp.float32),
    )(s1_p, xws, dis, b1, W2p)


def _tc_final(s2_p, hws, dis, b2, n, out_dim):
    def body(s2p_ref, hws_ref, dis_ref, b2_ref, out_ref):
        s2 = s2p_ref[0, :n, :out_dim] + s2p_ref[1, :n, :out_dim]
        out_ref[...] = ((s2 + hws_ref[:, :out_dim]) * dis_ref[...]
                        + b2_ref[...])

    return pl.pallas_call(
        body,
        out_shape=jax.ShapeDtypeStruct((n, out_dim), jnp.float32),
    )(s2_p, hws, dis, b2)


def kernel(x, edge_index, edge_weight, W1, b1, W2, b2):
    n, d = x.shape
    e = edge_index.shape[1]
    h = W1.shape[1]
    out_dim = W2.shape[1]
    w2pad = 16

    assert e % (NW * CHUNK) == 0
    cpw = e // (NW * CHUNK)
    assert cpw % NBUF == 0
    ei = edge_index.astype(jnp.int32).reshape(2, NW, cpw, CHUNK)
    ew_p = edge_weight.astype(jnp.float32).reshape(NW, cpw, CHUNK)

    deg_p = _make_deg_kernel(cpw)(ei, ew_p)
    xws, dis = _tc_prep(deg_p, x, W1, n)
    s1_p = _make_agg_kernel(cpw, h, n, False)(xws, ei, ew_p)
    W2p = jnp.concatenate(
        [W2, jnp.zeros((h, w2pad - out_dim), jnp.float32)], axis=1)
    hws = _tc_mid(s1_p, xws, dis, b1.reshape(1, h), W2p, n)
    s2_p = _make_agg_kernel(cpw, w2pad, n, True)(hws, ei, ew_p)
    return _tc_final(s2_p, hws, dis, b2.reshape(1, out_dim), n, out_dim)
